# baseline (speedup 1.0000x reference)
import functools
import math

import numpy as np
import jax
import jax.numpy as jnp
from jax import lax
from jax.experimental import pallas as pl
from jax.experimental.pallas import tpu as pltpu
from jax.experimental.pallas import tpu_sc as plsc

_NODE_SIZE = 100000
_NUM_SAMPLED = 64
_BATCH = 4096
_D = 128

_NC, _NS = 2, 16
_NW = _NC * _NS
_BPW = _BATCH // _NW
_NCHUNK = 8
_CHUNK = _BPW // _NCHUNK
_NBCHUNK = 4
_BCHUNK = _BPW // _NBCHUNK

_SAMPLED_NP = np.asarray([
    59469, 5933, 34593, 88, 1402, 1, 155, 45397, 0, 12, 134, 2, 11, 29, 9, 7,
    13, 88174, 5142, 1203, 3, 15480, 9736, 25, 4129, 213, 15, 8, 5, 3868,
    49816, 477, 75, 2088, 603, 1661, 1791, 4, 3224, 2876, 66, 296, 11158, 19,
    58866, 649, 53, 47, 16, 506, 33192, 26994, 1006, 81516, 1702, 59, 81,
    26363, 14833, 1021, 243, 22359, 7859, 30], np.int32)


@functools.lru_cache(maxsize=None)
def _make_sc_gather():
    mesh = plsc.VectorSubcoreMesh(core_axis_name="c", subcore_axis_name="s",
                                  num_cores=_NC, num_subcores=_NS)

    @functools.partial(
        pl.kernel,
        out_type=(jax.ShapeDtypeStruct((_BATCH, _D), jnp.float32),
                  jax.ShapeDtypeStruct((_BATCH,), jnp.float32)),
        mesh=mesh,
        scratch_types=[
            pltpu.VMEM((_BPW,), jnp.int32),
            pltpu.VMEM((_BPW, _D), jnp.float32),
            pltpu.VMEM((_BPW,), jnp.float32),
            pltpu.SemaphoreType.DMA,
            pltpu.SemaphoreType.DMA,
        ],
    )
    def sc_gather(lab_hbm, w_hbm, b_hbm, rows_out, bt_out,
                  idx_v, rows_v, bv_v, sem_w, sem_b):
        wid = lax.axis_index("s") * _NC + lax.axis_index("c")
        base = wid * _BPW
        pltpu.sync_copy(lab_hbm.at[pl.ds(base, _BPW)], idx_v)
        cpbs = [
            pltpu.async_copy(
                b_hbm.at[idx_v.at[pl.ds(j * _BCHUNK, _BCHUNK)]],
                bv_v.at[pl.ds(j * _BCHUNK, _BCHUNK)],
                sem_b)
            for j in range(_NBCHUNK)
        ]
        cps = [
            pltpu.async_copy(
                w_hbm.at[idx_v.at[pl.ds(j * _CHUNK, _CHUNK)]],
                rows_v.at[pl.ds(j * _CHUNK, _CHUNK)],
                sem_w)
            for j in range(_NCHUNK)
        ]
        for cp in cps:
            cp.wait()
        for cp in cpbs:
            cp.wait()
        pltpu.sync_copy(rows_v, rows_out.at[pl.ds(base, _BPW)])
        pltpu.sync_copy(bv_v, bt_out.at[pl.ds(base, _BPW)])

    return sc_gather


def _cand_body(emb_ref, lab_ref, samp_ref, boff_ref, w_any,
               ms_ref, ses_ref, sw_v, sem):
    cps = []
    for j in range(_NUM_SAMPLED):
        r = int(_SAMPLED_NP[j])
        cp = pltpu.make_async_copy(w_any.at[pl.ds(r, 1), :],
                                   sw_v.at[pl.ds(j, 1), :], sem)
        cp.start()
        cps.append(cp)
    for cp in cps:
        cp.wait()
    emb = emb_ref[...]
    sw = sw_v[...]
    slog = lax.dot_general(sw, emb, (((1,), (1,)), ((), ())),
                           preferred_element_type=jnp.float32)
    slog = slog + boff_ref[...]
    hit = samp_ref[...] == lab_ref[...]
    slog = jnp.where(hit, jnp.float32(-1e9), slog)
    ms = jnp.max(slog, axis=0, keepdims=True)
    ms_ref[...] = ms
    ses_ref[...] = jnp.sum(jnp.exp(slog - ms), axis=0, keepdims=True)


def _final_body(emb_ref, rows_ref, bt_ref, corr_ref, ms_ref, ses_ref,
                loss_ref):
    emb = emb_ref[...]
    tw = rows_ref[...]

    e2 = emb * tw
    ones = jnp.ones((8, _D), jnp.float32)
    td8 = lax.dot_general(ones, e2, (((1,), (1,)), ((), ())),
                          preferred_element_type=jnp.float32)
    td = td8[0:1, :]
    tl = td + bt_ref[...] - corr_ref[...]

    ms = ms_ref[...]
    m2 = jnp.maximum(ms, tl)
    se = ses_ref[...] * jnp.exp(ms - m2) + jnp.exp(tl - m2)
    li = m2 + jnp.log(se) - tl
    loss_ref[...] = jnp.sum(li, axis=1, keepdims=True) * (1.0 / _BATCH)


def kernel(labels, embed, w, b):
    lab_t = labels.reshape(1, _BATCH)
    samp = jnp.asarray(_SAMPLED_NP)
    samp_t = samp.reshape(_NUM_SAMPLED, 1)
    logn = jnp.log(float(_NODE_SIZE) + 1.0)
    labf = labels.astype(jnp.float32)
    p_lab = (jnp.log(labf + 2.0) - jnp.log(labf + 1.0)) / logn
    corr_t = jnp.log(-jnp.expm1(_NUM_SAMPLED * jnp.log1p(-p_lab)))
    corr_t = corr_t.reshape(1, _BATCH)
    sampf = samp.astype(jnp.float32)
    p_s = (jnp.log(sampf + 2.0) - jnp.log(sampf + 1.0)) / logn
    soff = jnp.log(-jnp.expm1(_NUM_SAMPLED * jnp.log1p(-p_s)))
    boff_t = (b[samp] - soff).reshape(_NUM_SAMPLED, 1)

    rows, bt = _make_sc_gather()(labels, w, b)

    ms, ses = pl.pallas_call(
        _cand_body,
        out_shape=(jax.ShapeDtypeStruct((1, _BATCH), jnp.float32),
                   jax.ShapeDtypeStruct((1, _BATCH), jnp.float32)),
        in_specs=[
            pl.BlockSpec(memory_space=pltpu.VMEM),
            pl.BlockSpec(memory_space=pltpu.VMEM),
            pl.BlockSpec(memory_space=pltpu.VMEM),
            pl.BlockSpec(memory_space=pltpu.VMEM),
            pl.BlockSpec(memory_space=pl.ANY),
        ],
        scratch_shapes=[
            pltpu.VMEM((_NUM_SAMPLED, _D), jnp.float32),
            pltpu.SemaphoreType.DMA,
        ],
    )(embed, lab_t, samp_t, boff_t, w)

    loss = pl.pallas_call(
        _final_body,
        out_shape=jax.ShapeDtypeStruct((1, 1), jnp.float32),
    )(embed, rows, bt.reshape(1, _BATCH), corr_t, ms, ses)
    return embed, loss.reshape(())


# speedup vs baseline: 1.0331x; 1.0331x over previous
import functools
import math

import numpy as np
import jax
import jax.numpy as jnp
from jax import lax
from jax.experimental import pallas as pl
from jax.experimental.pallas import tpu as pltpu
from jax.experimental.pallas import tpu_sc as plsc

_NODE_SIZE = 100000
_NUM_SAMPLED = 64
_BATCH = 4096
_D = 128

_NC, _NS = 2, 16
_NW = _NC * _NS
_BPW = _BATCH // _NW
_NCHUNK = 8
_CHUNK = _BPW // _NCHUNK
_NBCHUNK = 4
_BCHUNK = _BPW // _NBCHUNK

_SAMPLED_NP = np.asarray([
    59469, 5933, 34593, 88, 1402, 1, 155, 45397, 0, 12, 134, 2, 11, 29, 9, 7,
    13, 88174, 5142, 1203, 3, 15480, 9736, 25, 4129, 213, 15, 8, 5, 3868,
    49816, 477, 75, 2088, 603, 1661, 1791, 4, 3224, 2876, 66, 296, 11158, 19,
    58866, 649, 53, 47, 16, 506, 33192, 26994, 1006, 81516, 1702, 59, 81,
    26363, 14833, 1021, 243, 22359, 7859, 30], np.int32)


@functools.lru_cache(maxsize=None)
def _make_sc_gather():
    mesh = plsc.VectorSubcoreMesh(core_axis_name="c", subcore_axis_name="s",
                                  num_cores=_NC, num_subcores=_NS)

    @functools.partial(
        pl.kernel,
        out_type=(jax.ShapeDtypeStruct((_BATCH, _D), jnp.float32),
                  jax.ShapeDtypeStruct((_BATCH,), jnp.float32)),
        mesh=mesh,
        scratch_types=[
            pltpu.VMEM((_BPW,), jnp.int32),
            pltpu.VMEM((_BPW, _D), jnp.float32),
            pltpu.VMEM((_BPW,), jnp.float32),
            pltpu.SemaphoreType.DMA,
            pltpu.SemaphoreType.DMA,
        ],
    )
    def sc_gather(lab_hbm, w_hbm, b_hbm, rows_out, bt_out,
                  idx_v, rows_v, bv_v, sem_w, sem_b):
        wid = lax.axis_index("s") * _NC + lax.axis_index("c")
        base = wid * _BPW
        pltpu.sync_copy(lab_hbm.at[pl.ds(base, _BPW)], idx_v)
        cpbs = [
            pltpu.async_copy(
                b_hbm.at[idx_v.at[pl.ds(j * _BCHUNK, _BCHUNK)]],
                bv_v.at[pl.ds(j * _BCHUNK, _BCHUNK)],
                sem_b)
            for j in range(_NBCHUNK)
        ]
        cps = [
            pltpu.async_copy(
                w_hbm.at[idx_v.at[pl.ds(j * _CHUNK, _CHUNK)]],
                rows_v.at[pl.ds(j * _CHUNK, _CHUNK)],
                sem_w)
            for j in range(_NCHUNK)
        ]
        for cp in cps:
            cp.wait()
        for cp in cpbs:
            cp.wait()
        pltpu.sync_copy(rows_v, rows_out.at[pl.ds(base, _BPW)])
        pltpu.sync_copy(bv_v, bt_out.at[pl.ds(base, _BPW)])

    return sc_gather


def _cand_body(emb_ref, lab_ref, samp_ref, boff_ref, w_any,
               ms_ref, ses_ref, emb_out_ref, sw_v, sem):
    cps = []
    for j in range(_NUM_SAMPLED):
        r = int(_SAMPLED_NP[j])
        cp = pltpu.make_async_copy(w_any.at[pl.ds(r, 1), :],
                                   sw_v.at[pl.ds(j, 1), :], sem)
        cp.start()
        cps.append(cp)
    for cp in cps:
        cp.wait()
    emb = emb_ref[...]
    emb_out_ref[...] = emb
    sw = sw_v[...]
    slog = lax.dot_general(sw, emb, (((1,), (1,)), ((), ())),
                           preferred_element_type=jnp.float32)
    slog = slog + boff_ref[...]
    hit = samp_ref[...] == lab_ref[...]
    slog = jnp.where(hit, jnp.float32(-1e9), slog)
    ms = jnp.max(slog, axis=0, keepdims=True)
    ms_ref[...] = ms
    ses_ref[...] = jnp.sum(jnp.exp(slog - ms), axis=0, keepdims=True)


def _final_body(emb_ref, rows_ref, bt_ref, corr_ref, ms_ref, ses_ref,
                loss_ref):
    emb = emb_ref[...]
    tw = rows_ref[...]

    e2 = emb * tw
    ones = jnp.ones((8, _D), jnp.float32)
    td8 = lax.dot_general(ones, e2, (((1,), (1,)), ((), ())),
                          preferred_element_type=jnp.float32)
    td = td8[0:1, :]
    tl = td + bt_ref[...] - corr_ref[...]

    ms = ms_ref[...]
    m2 = jnp.maximum(ms, tl)
    se = ses_ref[...] * jnp.exp(ms - m2) + jnp.exp(tl - m2)
    li = m2 + jnp.log(se) - tl
    loss_ref[...] = jnp.sum(li, axis=1, keepdims=True) * (1.0 / _BATCH)


def kernel(labels, embed, w, b):
    lab_t = labels.reshape(1, _BATCH)
    samp = jnp.asarray(_SAMPLED_NP)
    samp_t = samp.reshape(_NUM_SAMPLED, 1)
    logn = jnp.log(float(_NODE_SIZE) + 1.0)
    labf = labels.astype(jnp.float32)
    p_lab = (jnp.log(labf + 2.0) - jnp.log(labf + 1.0)) / logn
    corr_t = jnp.log(-jnp.expm1(_NUM_SAMPLED * jnp.log1p(-p_lab)))
    corr_t = corr_t.reshape(1, _BATCH)
    sampf = samp.astype(jnp.float32)
    p_s = (jnp.log(sampf + 2.0) - jnp.log(sampf + 1.0)) / logn
    soff = jnp.log(-jnp.expm1(_NUM_SAMPLED * jnp.log1p(-p_s)))
    boff_t = (b[samp] - soff).reshape(_NUM_SAMPLED, 1)

    rows, bt = _make_sc_gather()(labels, w, b)

    ms, ses, emb_out = pl.pallas_call(
        _cand_body,
        out_shape=(jax.ShapeDtypeStruct((1, _BATCH), jnp.float32),
                   jax.ShapeDtypeStruct((1, _BATCH), jnp.float32),
                   jax.ShapeDtypeStruct((_BATCH, _D), jnp.float32)),
        in_specs=[
            pl.BlockSpec(memory_space=pltpu.VMEM),
            pl.BlockSpec(memory_space=pltpu.VMEM),
            pl.BlockSpec(memory_space=pltpu.VMEM),
            pl.BlockSpec(memory_space=pltpu.VMEM),
            pl.BlockSpec(memory_space=pl.ANY),
        ],
        scratch_shapes=[
            pltpu.VMEM((_NUM_SAMPLED, _D), jnp.float32),
            pltpu.SemaphoreType.DMA,
        ],
    )(embed, lab_t, samp_t, boff_t, w)

    loss = pl.pallas_call(
        _final_body,
        out_shape=jax.ShapeDtypeStruct((1, 1), jnp.float32),
    )(embed, rows, bt.reshape(1, _BATCH), corr_t, ms, ses)
    return emb_out, loss.reshape(())
